# baseline (speedup 1.0000x reference)
import jax
import jax.numpy as jnp
from jax import lax
from jax.experimental import pallas as pl
from jax.experimental.pallas import tpu as pltpu

B = 1024
K = 100000
EPS = 0.1
LAMB = max(5.0, 1500.0 / (1.0 + 0.1 * 1))
COEFF = 1.0 / (1.0 + LAMB)

CH = 2000
NCHUNK = K // CH
NSL = CH // 8
RPS = (B + NCHUNK - 1) // NCHUNK


def _window_copies(tgt_smem, cos_hbm, phi_hbm, cw_ref, pw_ref, sem_c, sem_p, i):
    t = tgt_smem[i]
    row8 = pl.multiple_of((t // 8) * 8, 8)
    col = pl.multiple_of((i // 128) * 128, 128)
    cp_c = pltpu.make_async_copy(
        cos_hbm.at[pl.ds(row8, 8), pl.ds(col, 128)], cw_ref.at[i], sem_c)
    cp_p = pltpu.make_async_copy(
        phi_hbm.at[pl.ds(row8, 8), pl.ds(col, 128)], pw_ref.at[i], sem_p)
    return cp_c, cp_p


def _tc_body(tgt_smem, x_ref, cos_hbm, phi_hbm, tgt_ref, out_ref,
             m_ref, s_ref, r_ref, ex_ref, cw_ref, pw_ref, sem_c, sem_p):
    c = pl.program_id(0)

    @pl.when(c == 0)
    def _init():
        m_ref[...] = jnp.full((1, B), -jnp.inf, jnp.float32)
        s_ref[...] = jnp.zeros((1, B), jnp.float32)
        r_ref[...] = jnp.zeros((1, B), jnp.float32)

    lo = c * RPS
    hi = jnp.minimum(lo + RPS, B)

    def _enq(i, carry):
        cp_c, cp_p = _window_copies(tgt_smem, cos_hbm, phi_hbm,
                                    cw_ref, pw_ref, sem_c, sem_p, i)
        cp_c.start()
        cp_p.start()
        return carry

    lax.fori_loop(lo, hi, _enq, 0)

    cmc = x_ref[0:8, :]
    for k in range(1, NSL):
        cmc = jnp.maximum(cmc, x_ref[k * 8:(k + 1) * 8, :])
    mb_old = m_ref[...]
    mb = jnp.maximum(mb_old, jnp.max(cmc, axis=0, keepdims=True))
    m_ref[...] = mb

    for k in range(NSL):
        xa = x_ref[k * 8:(k + 1) * 8, :]
        ex_ref[k * 8:(k + 1) * 8, :] = jnp.exp(xa - mb)
    ones = jnp.ones((1, CH), jnp.float32)
    s_ref[...] = (s_ref[...] * jnp.exp(mb_old - mb)
                  + jax.lax.dot_general(
                      ones, ex_ref[...], (((1,), (0,)), ((), ())),
                      preferred_element_type=jnp.float32))
    r_ref[...] = r_ref[...] + jax.lax.dot_general(
        ones, x_ref[...], (((1,), (0,)), ((), ())),
        preferred_element_type=jnp.float32)

    @pl.when(c == NCHUNK - 1)
    def _last():
        def _drain(i, carry):
            cp_c, cp_p = _window_copies(tgt_smem, cos_hbm, phi_hbm,
                                        cw_ref, pw_ref, sem_c, sem_p, i)
            cp_c.wait()
            cp_p.wait()
            return carry

        lax.fori_loop(0, B, _drain, 0)

        m_b = m_ref[...]
        s_b = s_ref[...]
        r_b = r_ref[...]
        m_all = jnp.transpose(m_b, (1, 0))
        s_all = jnp.transpose(s_b, (1, 0))
        r_all = jnp.transpose(r_b, (1, 0))

        tv = tgt_ref[...]
        iv = lax.broadcasted_iota(jnp.int32, (B, 1), 0)
        sub = (tv % 8).reshape(B, 1, 1)
        lane = (iv % 128).reshape(B, 1, 1)
        d1 = lax.broadcasted_iota(jnp.int32, (B, 8, 128), 1)
        d2 = lax.broadcasted_iota(jnp.int32, (B, 8, 128), 2)
        sel = jnp.logical_and(d1 == sub, d2 == lane)
        ct = jnp.sum(jnp.where(sel, cw_ref[...], 0.0), axis=(1, 2)).reshape(B, 1)
        pt = jnp.sum(jnp.where(sel, pw_ref[...], 0.0), axis=(1, 2)).reshape(B, 1)

        delta = (pt - ct) * COEFF
        ot = ct + delta
        m2 = jnp.maximum(m_all, ot)
        s2 = (s_all * jnp.exp(m_all - m2)
              + jnp.exp(ot - m2) - jnp.exp(ct - m2))
        lse = m2 + jnp.log(s2)
        per_row = ((1.0 - EPS) * (ot - lse)
                   + (EPS / K) * ((r_all + delta) - K * lse))
        out_ref[...] = -jnp.sum(per_row, keepdims=True) / B


_tc_loss = pl.pallas_call(
    _tc_body,
    grid_spec=pltpu.PrefetchScalarGridSpec(
        num_scalar_prefetch=1,
        grid=(NCHUNK,),
        in_specs=[
            pl.BlockSpec((CH, B), lambda c, tgt: (c, 0)),
            pl.BlockSpec(memory_space=pltpu.HBM),
            pl.BlockSpec(memory_space=pltpu.HBM),
            pl.BlockSpec((B, 1), lambda c, tgt: (0, 0)),
        ],
        out_specs=pl.BlockSpec((1, 1), lambda c, tgt: (0, 0)),
        scratch_shapes=[
            pltpu.VMEM((1, B), jnp.float32),
            pltpu.VMEM((1, B), jnp.float32),
            pltpu.VMEM((1, B), jnp.float32),
            pltpu.VMEM((CH, B), jnp.float32),
            pltpu.VMEM((B, 8, 128), jnp.float32),
            pltpu.VMEM((B, 8, 128), jnp.float32),
            pltpu.SemaphoreType.DMA,
            pltpu.SemaphoreType.DMA,
        ],
    ),
    out_shape=jax.ShapeDtypeStruct((1, 1), jnp.float32),
)


def kernel(cos_theta, phi_theta, targets):
    cos_t_view = cos_theta.T
    phi_t_view = phi_theta.T
    loss = _tc_loss(targets, cos_t_view, cos_t_view, phi_t_view,
                    targets.reshape(B, 1))
    return loss[0, 0]


# speedup vs baseline: 1.0066x; 1.0066x over previous
import jax
import jax.numpy as jnp
from jax import lax
from jax.experimental import pallas as pl
from jax.experimental.pallas import tpu as pltpu

B = 1024
K = 100000
EPS = 0.1
LAMB = max(5.0, 1500.0 / (1.0 + 0.1 * 1))
COEFF = 1.0 / (1.0 + LAMB)

CH = 2000
NCHUNK = K // CH
NSL = CH // 8
RPS = (B + NCHUNK - 1) // NCHUNK


def _window_copies(tgt_smem, cos_hbm, phi_hbm, cw_ref, pw_ref, sem_c, sem_p, i):
    t = tgt_smem[i]
    row8 = pl.multiple_of((t // 8) * 8, 8)
    col = pl.multiple_of((i // 128) * 128, 128)
    cp_c = pltpu.make_async_copy(
        cos_hbm.at[pl.ds(row8, 8), pl.ds(col, 128)], cw_ref.at[i], sem_c)
    cp_p = pltpu.make_async_copy(
        phi_hbm.at[pl.ds(row8, 8), pl.ds(col, 128)], pw_ref.at[i], sem_p)
    return cp_c, cp_p


def _tc_body(tgt_smem, x_ref, cos_hbm, phi_hbm, tgt_ref, out_ref,
             m_ref, s_ref, r_ref, cw_ref, pw_ref, sem_c, sem_p):
    c = pl.program_id(0)

    @pl.when(c == 0)
    def _init():
        m_ref[...] = jnp.full((8, B), -jnp.inf, jnp.float32)
        s_ref[...] = jnp.zeros((8, B), jnp.float32)
        r_ref[...] = jnp.zeros((1, B), jnp.float32)

    lo = c * RPS
    hi = jnp.minimum(lo + RPS, B)

    def _enq(i, carry):
        cp_c, cp_p = _window_copies(tgt_smem, cos_hbm, phi_hbm,
                                    cw_ref, pw_ref, sem_c, sem_p, i)
        cp_c.start()
        cp_p.start()
        return carry

    lax.fori_loop(lo, hi, _enq, 0)

    m_prev = m_ref[...]
    cm0, cm1 = m_prev, x_ref[8:16, :]
    for k in range(0, NSL, 2):
        cm0 = jnp.maximum(cm0, x_ref[k * 8:(k + 1) * 8, :])
    for k in range(3, NSL, 2):
        cm1 = jnp.maximum(cm1, x_ref[k * 8:(k + 1) * 8, :])
    cm = jnp.maximum(cm0, cm1)

    s = s_ref[...] * jnp.exp(m_prev - cm)
    for k in range(NSL):
        xa = x_ref[k * 8:(k + 1) * 8, :]
        s = s + jnp.exp(xa - cm)
    m_ref[...] = cm
    s_ref[...] = s
    r_ref[...] = r_ref[...] + jax.lax.dot_general(
        jnp.ones((1, CH), jnp.float32), x_ref[...],
        (((1,), (0,)), ((), ())), preferred_element_type=jnp.float32)

    @pl.when(c == NCHUNK - 1)
    def _last():
        def _drain(i, carry):
            cp_c, cp_p = _window_copies(tgt_smem, cos_hbm, phi_hbm,
                                        cw_ref, pw_ref, sem_c, sem_p, i)
            cp_c.wait()
            cp_p.wait()
            return carry

        lax.fori_loop(0, B, _drain, 0)

        m_b = jnp.max(cm, axis=0, keepdims=True)
        s_b = jnp.sum(s * jnp.exp(cm - m_b), axis=0, keepdims=True)
        r_b = r_ref[...]
        m_all = jnp.transpose(m_b, (1, 0))
        s_all = jnp.transpose(s_b, (1, 0))
        r_all = jnp.transpose(r_b, (1, 0))

        tv = tgt_ref[...]
        iv = lax.broadcasted_iota(jnp.int32, (B, 1), 0)
        sub = (tv % 8).reshape(B, 1, 1)
        lane = (iv % 128).reshape(B, 1, 1)
        d1 = lax.broadcasted_iota(jnp.int32, (B, 8, 128), 1)
        d2 = lax.broadcasted_iota(jnp.int32, (B, 8, 128), 2)
        sel = jnp.logical_and(d1 == sub, d2 == lane)
        ct = jnp.sum(jnp.where(sel, cw_ref[...], 0.0), axis=(1, 2)).reshape(B, 1)
        pt = jnp.sum(jnp.where(sel, pw_ref[...], 0.0), axis=(1, 2)).reshape(B, 1)

        delta = (pt - ct) * COEFF
        ot = ct + delta
        m2 = jnp.maximum(m_all, ot)
        s2 = (s_all * jnp.exp(m_all - m2)
              + jnp.exp(ot - m2) - jnp.exp(ct - m2))
        lse = m2 + jnp.log(s2)
        per_row = ((1.0 - EPS) * (ot - lse)
                   + (EPS / K) * ((r_all + delta) - K * lse))
        out_ref[...] = -jnp.sum(per_row, keepdims=True) / B


_tc_loss = pl.pallas_call(
    _tc_body,
    grid_spec=pltpu.PrefetchScalarGridSpec(
        num_scalar_prefetch=1,
        grid=(NCHUNK,),
        in_specs=[
            pl.BlockSpec((CH, B), lambda c, tgt: (c, 0)),
            pl.BlockSpec(memory_space=pltpu.HBM),
            pl.BlockSpec(memory_space=pltpu.HBM),
            pl.BlockSpec((B, 1), lambda c, tgt: (0, 0)),
        ],
        out_specs=pl.BlockSpec((1, 1), lambda c, tgt: (0, 0)),
        scratch_shapes=[
            pltpu.VMEM((8, B), jnp.float32),
            pltpu.VMEM((8, B), jnp.float32),
            pltpu.VMEM((1, B), jnp.float32),
            pltpu.VMEM((B, 8, 128), jnp.float32),
            pltpu.VMEM((B, 8, 128), jnp.float32),
            pltpu.SemaphoreType.DMA,
            pltpu.SemaphoreType.DMA,
        ],
    ),
    out_shape=jax.ShapeDtypeStruct((1, 1), jnp.float32),
)


def kernel(cos_theta, phi_theta, targets):
    cos_t_view = cos_theta.T
    phi_t_view = phi_theta.T
    loss = _tc_loss(targets, cos_t_view, cos_t_view, phi_t_view,
                    targets.reshape(B, 1))
    return loss[0, 0]


# speedup vs baseline: 1.0843x; 1.0772x over previous
import jax
import jax.numpy as jnp
from jax import lax
from jax.experimental import pallas as pl
from jax.experimental.pallas import tpu as pltpu

B = 1024
K = 100000
EPS = 0.1
LAMB = max(5.0, 1500.0 / (1.0 + 0.1 * 1))
COEFF = 1.0 / (1.0 + LAMB)

CH = 4000
NCHUNK = K // CH
NSL = CH // 8
RPS = (B + NCHUNK - 1) // NCHUNK


def _window_copies(tgt_smem, cos_hbm, phi_hbm, cw_ref, pw_ref, sem_c, sem_p, i):
    t = tgt_smem[i]
    row8 = pl.multiple_of((t // 8) * 8, 8)
    col = pl.multiple_of((i // 128) * 128, 128)
    cp_c = pltpu.make_async_copy(
        cos_hbm.at[pl.ds(row8, 8), pl.ds(col, 128)], cw_ref.at[i], sem_c)
    cp_p = pltpu.make_async_copy(
        phi_hbm.at[pl.ds(row8, 8), pl.ds(col, 128)], pw_ref.at[i], sem_p)
    return cp_c, cp_p


def _tc_body(tgt_smem, x_ref, cos_hbm, phi_hbm, tgt_ref, out_ref,
             m_ref, s_ref, r_ref, cw_ref, pw_ref, sem_c, sem_p):
    c = pl.program_id(0)

    @pl.when(c == 0)
    def _init():
        m_ref[...] = jnp.full((8, B), -jnp.inf, jnp.float32)
        s_ref[...] = jnp.zeros((8, B), jnp.float32)
        r_ref[...] = jnp.zeros((1, B), jnp.float32)

    lo = c * RPS
    hi = jnp.minimum(lo + RPS, B)

    def _enq(i, carry):
        cp_c, cp_p = _window_copies(tgt_smem, cos_hbm, phi_hbm,
                                    cw_ref, pw_ref, sem_c, sem_p, i)
        cp_c.start()
        cp_p.start()
        return carry

    lax.fori_loop(lo, hi, _enq, 0)

    m_prev = m_ref[...]
    cm0, cm1 = m_prev, x_ref[8:16, :]
    for k in range(0, NSL, 2):
        cm0 = jnp.maximum(cm0, x_ref[k * 8:(k + 1) * 8, :])
    for k in range(3, NSL, 2):
        cm1 = jnp.maximum(cm1, x_ref[k * 8:(k + 1) * 8, :])
    cm = jnp.maximum(cm0, cm1)

    s = s_ref[...] * jnp.exp(m_prev - cm)
    for k in range(NSL):
        xa = x_ref[k * 8:(k + 1) * 8, :]
        s = s + jnp.exp(xa - cm)
    m_ref[...] = cm
    s_ref[...] = s
    r_ref[...] = r_ref[...] + jax.lax.dot_general(
        jnp.ones((1, CH), jnp.float32), x_ref[...],
        (((1,), (0,)), ((), ())), preferred_element_type=jnp.float32)

    @pl.when(c == NCHUNK - 1)
    def _last():
        def _drain(i, carry):
            cp_c, cp_p = _window_copies(tgt_smem, cos_hbm, phi_hbm,
                                        cw_ref, pw_ref, sem_c, sem_p, i)
            cp_c.wait()
            cp_p.wait()
            return carry

        lax.fori_loop(0, B, _drain, 0)

        m_b = jnp.max(cm, axis=0, keepdims=True)
        s_b = jnp.sum(s * jnp.exp(cm - m_b), axis=0, keepdims=True)
        r_b = r_ref[...]
        m_all = jnp.transpose(m_b, (1, 0))
        s_all = jnp.transpose(s_b, (1, 0))
        r_all = jnp.transpose(r_b, (1, 0))

        tv = tgt_ref[...]
        iv = lax.broadcasted_iota(jnp.int32, (B, 1), 0)
        sub = (tv % 8).reshape(B, 1, 1)
        lane = (iv % 128).reshape(B, 1, 1)
        d1 = lax.broadcasted_iota(jnp.int32, (B, 8, 128), 1)
        d2 = lax.broadcasted_iota(jnp.int32, (B, 8, 128), 2)
        sel = jnp.logical_and(d1 == sub, d2 == lane)
        ct = jnp.sum(jnp.where(sel, cw_ref[...], 0.0), axis=(1, 2)).reshape(B, 1)
        pt = jnp.sum(jnp.where(sel, pw_ref[...], 0.0), axis=(1, 2)).reshape(B, 1)

        delta = (pt - ct) * COEFF
        ot = ct + delta
        m2 = jnp.maximum(m_all, ot)
        s2 = (s_all * jnp.exp(m_all - m2)
              + jnp.exp(ot - m2) - jnp.exp(ct - m2))
        lse = m2 + jnp.log(s2)
        per_row = ((1.0 - EPS) * (ot - lse)
                   + (EPS / K) * ((r_all + delta) - K * lse))
        out_ref[...] = -jnp.sum(per_row, keepdims=True) / B


_tc_loss = pl.pallas_call(
    _tc_body,
    grid_spec=pltpu.PrefetchScalarGridSpec(
        num_scalar_prefetch=1,
        grid=(NCHUNK,),
        in_specs=[
            pl.BlockSpec((CH, B), lambda c, tgt: (c, 0)),
            pl.BlockSpec(memory_space=pltpu.HBM),
            pl.BlockSpec(memory_space=pltpu.HBM),
            pl.BlockSpec((B, 1), lambda c, tgt: (0, 0)),
        ],
        out_specs=pl.BlockSpec((1, 1), lambda c, tgt: (0, 0)),
        scratch_shapes=[
            pltpu.VMEM((8, B), jnp.float32),
            pltpu.VMEM((8, B), jnp.float32),
            pltpu.VMEM((1, B), jnp.float32),
            pltpu.VMEM((B, 8, 128), jnp.float32),
            pltpu.VMEM((B, 8, 128), jnp.float32),
            pltpu.SemaphoreType.DMA,
            pltpu.SemaphoreType.DMA,
        ],
    ),
    out_shape=jax.ShapeDtypeStruct((1, 1), jnp.float32),
)


def kernel(cos_theta, phi_theta, targets):
    cos_t_view = cos_theta.T
    phi_t_view = phi_theta.T
    loss = _tc_loss(targets, cos_t_view, cos_t_view, phi_t_view,
                    targets.reshape(B, 1))
    return loss[0, 0]


# speedup vs baseline: 1.0961x; 1.0109x over previous
import jax
import jax.numpy as jnp
from jax import lax
from jax.experimental import pallas as pl
from jax.experimental.pallas import tpu as pltpu

B = 1024
K = 100000
EPS = 0.1
LAMB = max(5.0, 1500.0 / (1.0 + 0.1 * 1))
COEFF = 1.0 / (1.0 + LAMB)

CH = 5000
NCHUNK = K // CH
NSL = CH // 8
RPS = (B + NCHUNK - 1) // NCHUNK


def _window_copies(tgt_smem, cos_hbm, phi_hbm, cw_ref, pw_ref, sem_c, sem_p, i):
    t = tgt_smem[i]
    row8 = pl.multiple_of((t // 8) * 8, 8)
    col = pl.multiple_of((i // 128) * 128, 128)
    cp_c = pltpu.make_async_copy(
        cos_hbm.at[pl.ds(row8, 8), pl.ds(col, 128)], cw_ref.at[i], sem_c)
    cp_p = pltpu.make_async_copy(
        phi_hbm.at[pl.ds(row8, 8), pl.ds(col, 128)], pw_ref.at[i], sem_p)
    return cp_c, cp_p


def _tc_body(tgt_smem, x_ref, cos_hbm, phi_hbm, tgt_ref, out_ref,
             m_ref, s_ref, r_ref, cw_ref, pw_ref, sem_c, sem_p):
    c = pl.program_id(0)

    @pl.when(c == 0)
    def _init():
        m_ref[...] = jnp.full((8, B), -jnp.inf, jnp.float32)
        s_ref[...] = jnp.zeros((8, B), jnp.float32)
        r_ref[...] = jnp.zeros((1, B), jnp.float32)

    lo = c * RPS
    hi = jnp.minimum(lo + RPS, B)

    def _enq(i, carry):
        cp_c, cp_p = _window_copies(tgt_smem, cos_hbm, phi_hbm,
                                    cw_ref, pw_ref, sem_c, sem_p, i)
        cp_c.start()
        cp_p.start()
        return carry

    lax.fori_loop(lo, hi, _enq, 0)

    m_prev = m_ref[...]
    cm0, cm1 = m_prev, x_ref[8:16, :]
    for k in range(0, NSL, 2):
        cm0 = jnp.maximum(cm0, x_ref[k * 8:(k + 1) * 8, :])
    for k in range(3, NSL, 2):
        cm1 = jnp.maximum(cm1, x_ref[k * 8:(k + 1) * 8, :])
    cm = jnp.maximum(cm0, cm1)

    s = s_ref[...] * jnp.exp(m_prev - cm)
    for k in range(NSL):
        xa = x_ref[k * 8:(k + 1) * 8, :]
        s = s + jnp.exp(xa - cm)
    m_ref[...] = cm
    s_ref[...] = s
    r_ref[...] = r_ref[...] + jax.lax.dot_general(
        jnp.ones((1, CH), jnp.float32), x_ref[...],
        (((1,), (0,)), ((), ())), preferred_element_type=jnp.float32)

    @pl.when(c == NCHUNK - 1)
    def _last():
        def _drain(i, carry):
            cp_c, cp_p = _window_copies(tgt_smem, cos_hbm, phi_hbm,
                                        cw_ref, pw_ref, sem_c, sem_p, i)
            cp_c.wait()
            cp_p.wait()
            return carry

        lax.fori_loop(0, B, _drain, 0)

        m_b = jnp.max(cm, axis=0, keepdims=True)
        s_b = jnp.sum(s * jnp.exp(cm - m_b), axis=0, keepdims=True)
        r_b = r_ref[...]
        m_all = jnp.transpose(m_b, (1, 0))
        s_all = jnp.transpose(s_b, (1, 0))
        r_all = jnp.transpose(r_b, (1, 0))

        tv = tgt_ref[...]
        iv = lax.broadcasted_iota(jnp.int32, (B, 1), 0)
        sub = (tv % 8).reshape(B, 1, 1)
        lane = (iv % 128).reshape(B, 1, 1)
        d1 = lax.broadcasted_iota(jnp.int32, (B, 8, 128), 1)
        d2 = lax.broadcasted_iota(jnp.int32, (B, 8, 128), 2)
        sel = jnp.logical_and(d1 == sub, d2 == lane)
        ct = jnp.sum(jnp.where(sel, cw_ref[...], 0.0), axis=(1, 2)).reshape(B, 1)
        pt = jnp.sum(jnp.where(sel, pw_ref[...], 0.0), axis=(1, 2)).reshape(B, 1)

        delta = (pt - ct) * COEFF
        ot = ct + delta
        m2 = jnp.maximum(m_all, ot)
        s2 = (s_all * jnp.exp(m_all - m2)
              + jnp.exp(ot - m2) - jnp.exp(ct - m2))
        lse = m2 + jnp.log(s2)
        per_row = ((1.0 - EPS) * (ot - lse)
                   + (EPS / K) * ((r_all + delta) - K * lse))
        out_ref[...] = -jnp.sum(per_row, keepdims=True) / B


_tc_loss = pl.pallas_call(
    _tc_body,
    grid_spec=pltpu.PrefetchScalarGridSpec(
        num_scalar_prefetch=1,
        grid=(NCHUNK,),
        in_specs=[
            pl.BlockSpec((CH, B), lambda c, tgt: (c, 0)),
            pl.BlockSpec(memory_space=pltpu.HBM),
            pl.BlockSpec(memory_space=pltpu.HBM),
            pl.BlockSpec((B, 1), lambda c, tgt: (0, 0)),
        ],
        out_specs=pl.BlockSpec((1, 1), lambda c, tgt: (0, 0)),
        scratch_shapes=[
            pltpu.VMEM((8, B), jnp.float32),
            pltpu.VMEM((8, B), jnp.float32),
            pltpu.VMEM((1, B), jnp.float32),
            pltpu.VMEM((B, 8, 128), jnp.float32),
            pltpu.VMEM((B, 8, 128), jnp.float32),
            pltpu.SemaphoreType.DMA,
            pltpu.SemaphoreType.DMA,
        ],
    ),
    out_shape=jax.ShapeDtypeStruct((1, 1), jnp.float32),
)


def kernel(cos_theta, phi_theta, targets):
    cos_t_view = cos_theta.T
    phi_t_view = phi_theta.T
    loss = _tc_loss(targets, cos_t_view, cos_t_view, phi_t_view,
                    targets.reshape(B, 1))
    return loss[0, 0]
